# baseline (speedup 1.0000x reference)
import functools

import jax
import jax.numpy as jnp
from jax import lax
from jax.experimental import pallas as pl
from jax.experimental.pallas import tpu as pltpu
from jax.experimental.pallas import tpu_sc as plsc

F = 128
H = 128
G = 8
BM = 256
K = 128
N_TILES = 32
N_SUB = 16


def _pad_up(n, m):
    return (n + m - 1) // m * m


NB = 2


def _deg_kernel(npad, ch):
    rows_pt = npad // N_SUB
    mesh = plsc.VectorSubcoreMesh(core_axis_name="c", subcore_axis_name="s")
    scratch = [
        pltpu.VMEM((ch, K), jnp.int32),
        pltpu.VMEM((K, H), jnp.float32),
        pltpu.VMEM_SHARED((npad, H), jnp.float32),
    ] + [pltpu.SemaphoreType.DMA] * NB

    @functools.partial(
        pl.kernel,
        out_type=jax.ShapeDtypeStruct((2, npad, H), jnp.float32),
        mesh=mesh,
        scratch_types=scratch,
    )
    def deg(dst_hbm, ones_hbm, zeros_hbm, out_hbm, di_all, ones_v, acc,
            *ssem):
        c = lax.axis_index("c")
        s = lax.axis_index("s")
        tid = c * N_SUB + s

        pltpu.sync_copy(ones_hbm, ones_v)
        pltpu.sync_copy(dst_hbm.at[tid], di_all)
        pltpu.sync_copy(zeros_hbm, acc.at[pl.ds(s * rows_pt, rows_pt)])
        plsc.subcore_barrier()

        def group(g, carry):
            for b in range(NB):
                j = g * NB + b

                @pl.when(j >= NB)
                def _drain():
                    pltpu.make_async_copy(
                        ones_v, acc.at[di_all.at[0]], ssem[b]).wait()

                pltpu.async_copy(ones_v, acc.at[di_all.at[j]], ssem[b],
                                 add=True)
            return carry

        lax.fori_loop(0, ch // NB, group, 0)
        for b in range(NB):
            pltpu.make_async_copy(ones_v, acc.at[di_all.at[0]], ssem[b]).wait()
        plsc.subcore_barrier()
        pltpu.sync_copy(acc.at[pl.ds(s * rows_pt, rows_pt)],
                        out_hbm.at[c, pl.ds(s * rows_pt, rows_pt)])

    return deg


def _scatter_kernel(npad, ch):
    rows_pt = npad // N_SUB
    ng = ch // NB
    mesh = plsc.VectorSubcoreMesh(core_axis_name="c", subcore_axis_name="s")
    scratch = (
        [pltpu.VMEM((ch, K), jnp.int32),
         pltpu.VMEM((2, NB, K), jnp.int32)]
        + [pltpu.VMEM((K, H), jnp.float32)] * NB
        + [pltpu.VMEM_SHARED((npad, H), jnp.float32)]
        + [pltpu.SemaphoreType.DMA] * (2 * NB + 2)
    )

    @functools.partial(
        pl.kernel,
        out_type=jax.ShapeDtypeStruct((2, npad, H), jnp.float32),
        mesh=mesh,
        scratch_types=scratch,
    )
    def scat(y_hbm, src_hbm, dst_hbm, zeros_hbm, out_hbm, di_all, si_buf,
             *rest):
        rows = rest[:NB]
        acc = rest[NB]
        gsem = rest[NB + 1:2 * NB + 1]
        ssem = rest[2 * NB + 1:3 * NB + 1]
        isem = rest[3 * NB + 1:]
        c = lax.axis_index("c")
        s = lax.axis_index("s")
        tid = c * N_SUB + s

        pltpu.sync_copy(src_hbm.at[tid, pl.ds(0, NB)], si_buf.at[0])
        pltpu.sync_copy(dst_hbm.at[tid], di_all)
        pltpu.sync_copy(zeros_hbm, acc.at[pl.ds(s * rows_pt, rows_pt)])
        plsc.subcore_barrier()

        for b in range(NB):
            pltpu.async_copy(y_hbm.at[si_buf.at[0, b]], rows[b], gsem[b])
        pltpu.async_copy(src_hbm.at[tid, pl.ds(NB, NB)], si_buf.at[1],
                         isem[1])

        def big_group(gg, carry):
            for p in range(2):
                g = gg * 2 + p
                for b in range(NB):
                    j = g * NB + b
                    pltpu.make_async_copy(
                        y_hbm.at[si_buf.at[0, 0]], rows[b], gsem[b]).wait()
                    pltpu.async_copy(rows[b], acc.at[di_all.at[j]], ssem[b],
                                     add=True)

                    if b == NB - 1:
                        @pl.when(g + 2 < ng)
                        def _prefetch():
                            pltpu.async_copy(
                                src_hbm.at[tid, pl.ds((g + 2) * NB, NB)],
                                si_buf.at[p], isem[p])

                    pltpu.make_async_copy(
                        rows[b], acc.at[di_all.at[0]], ssem[b]).wait()

                    @pl.when(g + 1 < ng)
                    def _advance():
                        if b == 0:
                            pltpu.make_async_copy(
                                src_hbm.at[tid, pl.ds(0, NB)],
                                si_buf.at[1 - p], isem[1 - p]).wait()
                        pltpu.async_copy(y_hbm.at[si_buf.at[1 - p, b]],
                                         rows[b], gsem[b])
            return carry

        lax.fori_loop(0, ng // 2, big_group, 0)
        plsc.subcore_barrier()
        pltpu.sync_copy(acc.at[pl.ds(s * rows_pt, rows_pt)],
                        out_hbm.at[c, pl.ds(s * rows_pt, rows_pt)])

    return scat


def _dinv_of(d0, d1):
    return lax.rsqrt(d0[:, 0:1] + d1[:, 0:1] + 1.0)


def _dot(a, b):
    return jnp.dot(a, b, preferred_element_type=jnp.float32,
                   precision=lax.Precision.HIGHEST)


def _t0_body(x_ref, w_ref, d0_ref, d1_ref, y_ref):
    dinv = _dinv_of(d0_ref[...], d1_ref[...])
    y_ref[...] = _dot(x_ref[...], w_ref[...]) * dinv


def _tmid_body(p0_ref, p1_ref, yp_ref, d0_ref, d1_ref, w_ref,
               rm_ref, rv_ref, ga_ref, be_ref, bc_ref, y_ref):
    dinv = _dinv_of(d0_ref[...], d1_ref[...])
    agg = (p0_ref[...] + p1_ref[...] + yp_ref[...]) * dinv + bc_ref[...]
    hb = (agg - rm_ref[...]) * lax.rsqrt(rv_ref[...] + 1e-5) * ga_ref[...] \
        + be_ref[...]
    h = jnp.maximum(hb, 0.0)
    y_ref[...] = _dot(h, w_ref[...]) * dinv


def _tfin_body(p0_ref, p1_ref, yp_ref, d0_ref, d1_ref, b_ref,
               rm_ref, rv_ref, ga_ref, be_ref, bc_ref,
               w1_ref, b1_ref, w2_ref, b2_ref, w3_ref, b3_ref,
               out_ref, pool_acc, cnt_acc):
    i = pl.program_id(0)
    nsteps = pl.num_programs(0)

    @pl.when(i == 0)
    def _init():
        pool_acc[...] = jnp.zeros_like(pool_acc)
        cnt_acc[...] = jnp.zeros_like(cnt_acc)

    dinv = _dinv_of(d0_ref[...], d1_ref[...])
    agg = (p0_ref[...] + p1_ref[...] + yp_ref[...]) * dinv + bc_ref[...]
    hb = (agg - rm_ref[...]) * lax.rsqrt(rv_ref[...] + 1e-5) * ga_ref[...] \
        + be_ref[...]
    h = jnp.maximum(hb, 0.0)

    gids = lax.broadcasted_iota(jnp.int32, (G, BM), 0)
    onehot = (b_ref[...] == gids).astype(jnp.float32)
    pool_acc[...] += _dot(onehot, h)
    cnt_acc[...] += jnp.broadcast_to(
        jnp.sum(onehot, axis=1, keepdims=True), (G, H))

    @pl.when(i == nsteps - 1)
    def _finish():
        pooled = pool_acc[...] / jnp.maximum(cnt_acc[...], 1.0)
        z = jnp.maximum(_dot(pooled, w1_ref[...]) + b1_ref[...], 0.0)
        z = jnp.maximum(_dot(z, w2_ref[...]) + b2_ref[...], 0.0)
        out_ref[...] = _dot(z, w3_ref[...]) + b3_ref[...]


def _row_spec(w):
    return pl.BlockSpec((BM, w), lambda i: (i, 0))


def _full_spec(r, c):
    return pl.BlockSpec((r, c), lambda i: (0, 0))


def kernel(x, edge_index, batch, Wc, bc, gamma, beta, rm, rv,
           Wm1, bm1, Wm2, bm2, Wm3, bm3):
    n, f = x.shape
    e = edge_index.shape[1]
    npad = _pad_up(n + 1, BM)
    ep = _pad_up(e, N_TILES * K * NB * 2)
    ch = ep // (N_TILES * K)

    xp = jnp.zeros((npad, f), jnp.float32).at[:n].set(x)
    srcp = jnp.concatenate(
        [edge_index[0].astype(jnp.int32),
         jnp.zeros((ep - e,), jnp.int32)]).reshape(N_TILES, ch, K)
    dstp = jnp.concatenate(
        [edge_index[1].astype(jnp.int32),
         jnp.full((ep - e,), n, jnp.int32)]).reshape(N_TILES, ch, K)
    bpad = jnp.full((1, npad), G, jnp.int32).at[0, :n].set(
        batch.astype(jnp.int32))

    zeros_w = jnp.zeros((npad // N_SUB, H), jnp.float32)
    ones_w = jnp.ones((K, H), jnp.float32)

    w2p = jnp.zeros((H, H), jnp.float32).at[:, :Wm2.shape[1]].set(Wm2)
    b2p = jnp.zeros((1, H), jnp.float32).at[0, :Wm2.shape[1]].set(bm2)
    w3p = jnp.zeros((H, H), jnp.float32).at[:Wm3.shape[0], 0].set(Wm3[:, 0])
    b3p = jnp.zeros((1, H), jnp.float32).at[0, 0].set(bm3[0])
    w1p = Wm1
    b1p = bm1.reshape(1, H)

    params = [(rm[i].reshape(1, H), rv[i].reshape(1, H),
               gamma[i].reshape(1, H), beta[i].reshape(1, H),
               bc[i].reshape(1, H)) for i in range(3)]

    grid = (npad // BM,)

    degp = _deg_kernel(npad, ch)(dstp, ones_w, zeros_w)
    d0, d1 = degp[0], degp[1]

    y = pl.pallas_call(
        _t0_body,
        grid=grid,
        in_specs=[_row_spec(f), _full_spec(f, H), _row_spec(H), _row_spec(H)],
        out_specs=_row_spec(H),
        out_shape=jax.ShapeDtypeStruct((npad, H), jnp.float32),
    )(xp, Wc[0], d0, d1)

    for i in range(2):
        part = _scatter_kernel(npad, ch)(y, srcp, dstp, zeros_w)
        rm_i, rv_i, ga_i, be_i, bc_i = params[i]
        y = pl.pallas_call(
            _tmid_body,
            grid=grid,
            in_specs=[_row_spec(H), _row_spec(H), _row_spec(H),
                      _row_spec(H), _row_spec(H), _full_spec(H, H),
                      _full_spec(1, H), _full_spec(1, H), _full_spec(1, H),
                      _full_spec(1, H), _full_spec(1, H)],
            out_specs=_row_spec(H),
            out_shape=jax.ShapeDtypeStruct((npad, H), jnp.float32),
        )(part[0], part[1], y, d0, d1, Wc[i + 1],
          rm_i, rv_i, ga_i, be_i, bc_i)

    part = _scatter_kernel(npad, ch)(y, srcp, dstp, zeros_w)
    rm_i, rv_i, ga_i, be_i, bc_i = params[2]
    out = pl.pallas_call(
        _tfin_body,
        grid=grid,
        in_specs=[_row_spec(H), _row_spec(H), _row_spec(H),
                  _row_spec(H), _row_spec(H),
                  pl.BlockSpec((1, BM), lambda i: (0, i)),
                  _full_spec(1, H), _full_spec(1, H), _full_spec(1, H),
                  _full_spec(1, H), _full_spec(1, H),
                  _full_spec(H, H), _full_spec(1, H), _full_spec(H, H),
                  _full_spec(1, H), _full_spec(H, H), _full_spec(1, H)],
        out_specs=_full_spec(G, H),
        out_shape=jax.ShapeDtypeStruct((G, H), jnp.float32),
        scratch_shapes=[pltpu.VMEM((G, H), jnp.float32),
                        pltpu.VMEM((G, H), jnp.float32)],
    )(part[0], part[1], y, d0, d1, bpad,
      rm_i, rv_i, ga_i, be_i, bc_i,
      w1p, b1p, w2p, b2p, w3p, b3p)

    return out[:, 0]


# speedup vs baseline: 1.0488x; 1.0488x over previous
import functools

import jax
import jax.numpy as jnp
from jax import lax
from jax.experimental import pallas as pl
from jax.experimental.pallas import tpu as pltpu
from jax.experimental.pallas import tpu_sc as plsc

F = 128
H = 128
G = 8
BM = 256
K = 128
N_TILES = 32
N_SUB = 16


def _pad_up(n, m):
    return (n + m - 1) // m * m


NB = 2
CH0_SHARE = 0.2


def _deg_kernel(npad, ch):
    rows_pt = npad // N_SUB
    mesh = plsc.VectorSubcoreMesh(core_axis_name="c", subcore_axis_name="s")
    scratch = [
        pltpu.VMEM((ch, K), jnp.int32),
        pltpu.VMEM((K, H), jnp.float32),
        pltpu.VMEM_SHARED((npad, H), jnp.float32),
    ] + [pltpu.SemaphoreType.DMA] * NB

    @functools.partial(
        pl.kernel,
        out_type=jax.ShapeDtypeStruct((2, npad, H), jnp.float32),
        mesh=mesh,
        scratch_types=scratch,
    )
    def deg(dst_hbm, ones_hbm, zeros_hbm, out_hbm, di_all, ones_v, acc,
            *ssem):
        c = lax.axis_index("c")
        s = lax.axis_index("s")
        tid = c * N_SUB + s

        pltpu.sync_copy(ones_hbm, ones_v)
        pltpu.sync_copy(dst_hbm.at[pl.ds(tid * ch, ch)], di_all)
        pltpu.sync_copy(zeros_hbm, acc.at[pl.ds(s * rows_pt, rows_pt)])
        plsc.subcore_barrier()

        def group(g, carry):
            for b in range(NB):
                j = g * NB + b

                @pl.when(j >= NB)
                def _drain():
                    pltpu.make_async_copy(
                        ones_v, acc.at[di_all.at[0]], ssem[b]).wait()

                pltpu.async_copy(ones_v, acc.at[di_all.at[j]], ssem[b],
                                 add=True)
            return carry

        lax.fori_loop(0, ch // NB, group, 0)
        for b in range(NB):
            pltpu.make_async_copy(ones_v, acc.at[di_all.at[0]], ssem[b]).wait()
        plsc.subcore_barrier()
        pltpu.sync_copy(acc.at[pl.ds(s * rows_pt, rows_pt)],
                        out_hbm.at[c, pl.ds(s * rows_pt, rows_pt)])

    return deg


def _scatter_kernel(npad, ch0, ch1):
    rows_pt = npad // N_SUB
    mesh = plsc.VectorSubcoreMesh(core_axis_name="c", subcore_axis_name="s")
    scratch = (
        [pltpu.VMEM((2, NB, K), jnp.int32)] * 2
        + [pltpu.VMEM((K, H), jnp.float32)] * NB
        + [pltpu.VMEM_SHARED((npad, H), jnp.float32)]
        + [pltpu.SemaphoreType.DMA] * (2 * NB + 4)
    )

    @functools.partial(
        pl.kernel,
        out_type=jax.ShapeDtypeStruct((2, npad, H), jnp.float32),
        mesh=mesh,
        scratch_types=scratch,
    )
    def scat(y_hbm, src_hbm, dst_hbm, zeros_hbm, out_hbm, si_buf, di_buf,
             *rest):
        rows = rest[:NB]
        acc = rest[NB]
        gsem = rest[NB + 1:2 * NB + 1]
        ssem = rest[2 * NB + 1:3 * NB + 1]
        isem = rest[3 * NB + 1:3 * NB + 3]
        jsem = rest[3 * NB + 3:]
        c = lax.axis_index("c")
        s = lax.axis_index("s")
        base = jnp.where(c == 0, s * ch0, N_SUB * ch0 + s * ch1)
        ng = jnp.where(c == 0, ch0 // NB, ch1 // NB)

        pltpu.sync_copy(src_hbm.at[pl.ds(base, NB)], si_buf.at[0])
        pltpu.sync_copy(dst_hbm.at[pl.ds(base, NB)], di_buf.at[0])
        pltpu.sync_copy(zeros_hbm, acc.at[pl.ds(s * rows_pt, rows_pt)])
        plsc.subcore_barrier()

        for b in range(NB):
            pltpu.async_copy(y_hbm.at[si_buf.at[0, b]], rows[b], gsem[b])
        pltpu.async_copy(src_hbm.at[pl.ds(base + NB, NB)], si_buf.at[1],
                         isem[1])
        pltpu.async_copy(dst_hbm.at[pl.ds(base + NB, NB)], di_buf.at[1],
                         jsem[1])

        def big_group(gg, carry):
            for p in range(2):
                g = gg * 2 + p
                for b in range(NB):
                    pltpu.make_async_copy(
                        y_hbm.at[si_buf.at[0, 0]], rows[b], gsem[b]).wait()
                    pltpu.async_copy(rows[b], acc.at[di_buf.at[p, b]],
                                     ssem[b], add=True)
                    pltpu.make_async_copy(
                        rows[b], acc.at[di_buf.at[0, 0]], ssem[b]).wait()

                    if b == NB - 1:
                        @pl.when(g + 2 < ng)
                        def _prefetch():
                            pltpu.async_copy(
                                src_hbm.at[pl.ds(base + (g + 2) * NB, NB)],
                                si_buf.at[p], isem[p])
                            pltpu.async_copy(
                                dst_hbm.at[pl.ds(base + (g + 2) * NB, NB)],
                                di_buf.at[p], jsem[p])

                    @pl.when(g + 1 < ng)
                    def _advance():
                        if b == 0:
                            pltpu.make_async_copy(
                                src_hbm.at[pl.ds(base, NB)],
                                si_buf.at[1 - p], isem[1 - p]).wait()
                            pltpu.make_async_copy(
                                dst_hbm.at[pl.ds(base, NB)],
                                di_buf.at[1 - p], jsem[1 - p]).wait()
                        pltpu.async_copy(y_hbm.at[si_buf.at[1 - p, b]],
                                         rows[b], gsem[b])
            return carry

        lax.fori_loop(0, lax.div(ng, 2), big_group, 0)
        plsc.subcore_barrier()
        pltpu.sync_copy(acc.at[pl.ds(s * rows_pt, rows_pt)],
                        out_hbm.at[c, pl.ds(s * rows_pt, rows_pt)])

    return scat


def _dinv_of(d0, d1):
    return lax.rsqrt(d0[:, 0:1] + d1[:, 0:1] + 1.0)


def _dot(a, b):
    return jnp.dot(a, b, preferred_element_type=jnp.float32,
                   precision=lax.Precision.HIGHEST)


def _t0_body(x_ref, w_ref, d0_ref, d1_ref, y_ref):
    dinv = _dinv_of(d0_ref[...], d1_ref[...])
    y_ref[...] = _dot(x_ref[...], w_ref[...]) * dinv


def _tmid_body(p0_ref, p1_ref, yp_ref, d0_ref, d1_ref, w_ref,
               rm_ref, rv_ref, ga_ref, be_ref, bc_ref, y_ref):
    dinv = _dinv_of(d0_ref[...], d1_ref[...])
    agg = (p0_ref[...] + p1_ref[...] + yp_ref[...]) * dinv + bc_ref[...]
    hb = (agg - rm_ref[...]) * lax.rsqrt(rv_ref[...] + 1e-5) * ga_ref[...] \
        + be_ref[...]
    h = jnp.maximum(hb, 0.0)
    y_ref[...] = _dot(h, w_ref[...]) * dinv


def _tfin_body(p0_ref, p1_ref, yp_ref, d0_ref, d1_ref, b_ref,
               rm_ref, rv_ref, ga_ref, be_ref, bc_ref,
               w1_ref, b1_ref, w2_ref, b2_ref, w3_ref, b3_ref,
               out_ref, pool_acc, cnt_acc):
    i = pl.program_id(0)
    nsteps = pl.num_programs(0)

    @pl.when(i == 0)
    def _init():
        pool_acc[...] = jnp.zeros_like(pool_acc)
        cnt_acc[...] = jnp.zeros_like(cnt_acc)

    dinv = _dinv_of(d0_ref[...], d1_ref[...])
    agg = (p0_ref[...] + p1_ref[...] + yp_ref[...]) * dinv + bc_ref[...]
    hb = (agg - rm_ref[...]) * lax.rsqrt(rv_ref[...] + 1e-5) * ga_ref[...] \
        + be_ref[...]
    h = jnp.maximum(hb, 0.0)

    gids = lax.broadcasted_iota(jnp.int32, (G, BM), 0)
    onehot = (b_ref[...] == gids).astype(jnp.float32)
    pool_acc[...] += _dot(onehot, h)
    cnt_acc[...] += jnp.broadcast_to(
        jnp.sum(onehot, axis=1, keepdims=True), (G, H))

    @pl.when(i == nsteps - 1)
    def _finish():
        pooled = pool_acc[...] / jnp.maximum(cnt_acc[...], 1.0)
        z = jnp.maximum(_dot(pooled, w1_ref[...]) + b1_ref[...], 0.0)
        z = jnp.maximum(_dot(z, w2_ref[...]) + b2_ref[...], 0.0)
        out_ref[...] = _dot(z, w3_ref[...]) + b3_ref[...]


def _row_spec(w):
    return pl.BlockSpec((BM, w), lambda i: (i, 0))


def _full_spec(r, c):
    return pl.BlockSpec((r, c), lambda i: (0, 0))


def kernel(x, edge_index, batch, Wc, bc, gamma, beta, rm, rv,
           Wm1, bm1, Wm2, bm2, Wm3, bm3):
    n, f = x.shape
    e = edge_index.shape[1]
    npad = _pad_up(n + 1, BM)
    ep = _pad_up(e, N_TILES * K * NB * 2)
    ch = ep // (N_TILES * K)
    ch_pair = ep // (N_SUB * K)
    ch0 = _pad_up(int(ch_pair * CH0_SHARE), 2 * NB)
    ch1 = ch_pair - ch0

    xp = jnp.zeros((npad, f), jnp.float32).at[:n].set(x)
    srcp = jnp.concatenate(
        [edge_index[0].astype(jnp.int32),
         jnp.zeros((ep - e,), jnp.int32)]).reshape(ep // K, K)
    dstp = jnp.concatenate(
        [edge_index[1].astype(jnp.int32),
         jnp.full((ep - e,), n, jnp.int32)]).reshape(ep // K, K)
    bpad = jnp.full((1, npad), G, jnp.int32).at[0, :n].set(
        batch.astype(jnp.int32))

    zeros_w = jnp.zeros((npad // N_SUB, H), jnp.float32)
    ones_w = jnp.ones((K, H), jnp.float32)

    w2p = jnp.zeros((H, H), jnp.float32).at[:, :Wm2.shape[1]].set(Wm2)
    b2p = jnp.zeros((1, H), jnp.float32).at[0, :Wm2.shape[1]].set(bm2)
    w3p = jnp.zeros((H, H), jnp.float32).at[:Wm3.shape[0], 0].set(Wm3[:, 0])
    b3p = jnp.zeros((1, H), jnp.float32).at[0, 0].set(bm3[0])
    w1p = Wm1
    b1p = bm1.reshape(1, H)

    params = [(rm[i].reshape(1, H), rv[i].reshape(1, H),
               gamma[i].reshape(1, H), beta[i].reshape(1, H),
               bc[i].reshape(1, H)) for i in range(3)]

    grid = (npad // BM,)

    degp = _deg_kernel(npad, ch)(dstp, ones_w, zeros_w)
    d0, d1 = degp[0], degp[1]

    y = pl.pallas_call(
        _t0_body,
        grid=grid,
        in_specs=[_row_spec(f), _full_spec(f, H), _row_spec(H), _row_spec(H)],
        out_specs=_row_spec(H),
        out_shape=jax.ShapeDtypeStruct((npad, H), jnp.float32),
    )(xp, Wc[0], d0, d1)

    for i in range(2):
        part = _scatter_kernel(npad, ch0, ch1)(y, srcp, dstp, zeros_w)
        rm_i, rv_i, ga_i, be_i, bc_i = params[i]
        y = pl.pallas_call(
            _tmid_body,
            grid=grid,
            in_specs=[_row_spec(H), _row_spec(H), _row_spec(H),
                      _row_spec(H), _row_spec(H), _full_spec(H, H),
                      _full_spec(1, H), _full_spec(1, H), _full_spec(1, H),
                      _full_spec(1, H), _full_spec(1, H)],
            out_specs=_row_spec(H),
            out_shape=jax.ShapeDtypeStruct((npad, H), jnp.float32),
        )(part[0], part[1], y, d0, d1, Wc[i + 1],
          rm_i, rv_i, ga_i, be_i, bc_i)

    part = _scatter_kernel(npad, ch0, ch1)(y, srcp, dstp, zeros_w)
    rm_i, rv_i, ga_i, be_i, bc_i = params[2]
    out = pl.pallas_call(
        _tfin_body,
        grid=grid,
        in_specs=[_row_spec(H), _row_spec(H), _row_spec(H),
                  _row_spec(H), _row_spec(H),
                  pl.BlockSpec((1, BM), lambda i: (0, i)),
                  _full_spec(1, H), _full_spec(1, H), _full_spec(1, H),
                  _full_spec(1, H), _full_spec(1, H),
                  _full_spec(H, H), _full_spec(1, H), _full_spec(H, H),
                  _full_spec(1, H), _full_spec(H, H), _full_spec(1, H)],
        out_specs=_full_spec(G, H),
        out_shape=jax.ShapeDtypeStruct((G, H), jnp.float32),
        scratch_shapes=[pltpu.VMEM((G, H), jnp.float32),
                        pltpu.VMEM((G, H), jnp.float32)],
    )(part[0], part[1], y, d0, d1, bpad,
      rm_i, rv_i, ga_i, be_i, bc_i,
      w1p, b1p, w2p, b2p, w3p, b3p)

    return out[:, 0]


# speedup vs baseline: 1.1367x; 1.0838x over previous
import functools

import jax
import jax.numpy as jnp
from jax import lax
from jax.experimental import pallas as pl
from jax.experimental.pallas import tpu as pltpu
from jax.experimental.pallas import tpu_sc as plsc

F = 128
H = 128
G = 8
BM = 256
K = 128
N_TILES = 32
N_SUB = 16


def _pad_up(n, m):
    return (n + m - 1) // m * m


NB = 2
CH0_SHARE = 0.8


def _deg_kernel(npad, ch):
    rows_pt = npad // N_SUB
    mesh = plsc.VectorSubcoreMesh(core_axis_name="c", subcore_axis_name="s")
    scratch = [
        pltpu.VMEM((ch, K), jnp.int32),
        pltpu.VMEM((K, H), jnp.float32),
        pltpu.VMEM_SHARED((npad, H), jnp.float32),
    ] + [pltpu.SemaphoreType.DMA] * NB

    @functools.partial(
        pl.kernel,
        out_type=jax.ShapeDtypeStruct((2, npad, H), jnp.float32),
        mesh=mesh,
        scratch_types=scratch,
    )
    def deg(dst_hbm, ones_hbm, zeros_hbm, out_hbm, di_all, ones_v, acc,
            *ssem):
        c = lax.axis_index("c")
        s = lax.axis_index("s")
        tid = c * N_SUB + s

        pltpu.sync_copy(ones_hbm, ones_v)
        pltpu.sync_copy(dst_hbm.at[pl.ds(tid * ch, ch)], di_all)
        pltpu.sync_copy(zeros_hbm, acc.at[pl.ds(s * rows_pt, rows_pt)])
        plsc.subcore_barrier()

        def group(g, carry):
            for b in range(NB):
                j = g * NB + b

                @pl.when(j >= NB)
                def _drain():
                    pltpu.make_async_copy(
                        ones_v, acc.at[di_all.at[0]], ssem[b]).wait()

                pltpu.async_copy(ones_v, acc.at[di_all.at[j]], ssem[b],
                                 add=True)
            return carry

        lax.fori_loop(0, ch // NB, group, 0)
        for b in range(NB):
            pltpu.make_async_copy(ones_v, acc.at[di_all.at[0]], ssem[b]).wait()
        plsc.subcore_barrier()
        pltpu.sync_copy(acc.at[pl.ds(s * rows_pt, rows_pt)],
                        out_hbm.at[c, pl.ds(s * rows_pt, rows_pt)])

    return deg


def _scatter_kernel(npad, ch0, ch1):
    rows_pt = npad // N_SUB
    mesh = plsc.VectorSubcoreMesh(core_axis_name="c", subcore_axis_name="s")
    scratch = (
        [pltpu.VMEM((2, NB, K), jnp.int32)] * 2
        + [pltpu.VMEM((K, H), jnp.float32)] * NB
        + [pltpu.VMEM_SHARED((npad, H), jnp.float32)]
        + [pltpu.SemaphoreType.DMA] * (2 * NB + 4)
    )

    @functools.partial(
        pl.kernel,
        out_type=jax.ShapeDtypeStruct((2, npad, H), jnp.float32),
        mesh=mesh,
        scratch_types=scratch,
    )
    def scat(y_hbm, src_hbm, dst_hbm, zeros_hbm, out_hbm, si_buf, di_buf,
             *rest):
        rows = rest[:NB]
        acc = rest[NB]
        gsem = rest[NB + 1:2 * NB + 1]
        ssem = rest[2 * NB + 1:3 * NB + 1]
        isem = rest[3 * NB + 1:3 * NB + 3]
        jsem = rest[3 * NB + 3:]
        c = lax.axis_index("c")
        s = lax.axis_index("s")
        base = jnp.where(c == 0, s * ch0, N_SUB * ch0 + s * ch1)
        ng = jnp.where(c == 0, ch0 // NB, ch1 // NB)

        pltpu.sync_copy(src_hbm.at[pl.ds(base, NB)], si_buf.at[0])
        pltpu.sync_copy(dst_hbm.at[pl.ds(base, NB)], di_buf.at[0])
        pltpu.sync_copy(zeros_hbm, acc.at[pl.ds(s * rows_pt, rows_pt)])
        plsc.subcore_barrier()

        for b in range(NB):
            pltpu.async_copy(y_hbm.at[si_buf.at[0, b]], rows[b], gsem[b])
        pltpu.async_copy(src_hbm.at[pl.ds(base + NB, NB)], si_buf.at[1],
                         isem[1])
        pltpu.async_copy(dst_hbm.at[pl.ds(base + NB, NB)], di_buf.at[1],
                         jsem[1])

        def big_group(gg, carry):
            for p in range(2):
                g = gg * 2 + p
                for b in range(NB):
                    pltpu.make_async_copy(
                        y_hbm.at[si_buf.at[0, 0]], rows[b], gsem[b]).wait()
                    pltpu.async_copy(rows[b], acc.at[di_buf.at[p, b]],
                                     ssem[b], add=True)
                    pltpu.make_async_copy(
                        rows[b], acc.at[di_buf.at[0, 0]], ssem[b]).wait()

                    if b == NB - 1:
                        @pl.when(g + 2 < ng)
                        def _prefetch():
                            pltpu.async_copy(
                                src_hbm.at[pl.ds(base + (g + 2) * NB, NB)],
                                si_buf.at[p], isem[p])
                            pltpu.async_copy(
                                dst_hbm.at[pl.ds(base + (g + 2) * NB, NB)],
                                di_buf.at[p], jsem[p])

                    @pl.when(g + 1 < ng)
                    def _advance():
                        if b == 0:
                            pltpu.make_async_copy(
                                src_hbm.at[pl.ds(base, NB)],
                                si_buf.at[1 - p], isem[1 - p]).wait()
                            pltpu.make_async_copy(
                                dst_hbm.at[pl.ds(base, NB)],
                                di_buf.at[1 - p], jsem[1 - p]).wait()
                        pltpu.async_copy(y_hbm.at[si_buf.at[1 - p, b]],
                                         rows[b], gsem[b])
            return carry

        lax.fori_loop(0, lax.div(ng, 2), big_group, 0)
        plsc.subcore_barrier()
        pltpu.sync_copy(acc.at[pl.ds(s * rows_pt, rows_pt)],
                        out_hbm.at[c, pl.ds(s * rows_pt, rows_pt)])

    return scat


def _dinv_of(d0, d1):
    return lax.rsqrt(d0[:, 0:1] + d1[:, 0:1] + 1.0)


def _dot(a, b):
    return jnp.dot(a, b, preferred_element_type=jnp.float32,
                   precision=lax.Precision.HIGHEST)


def _t0_body(x_ref, w_ref, d0_ref, d1_ref, y_ref):
    dinv = _dinv_of(d0_ref[...], d1_ref[...])
    y_ref[...] = _dot(x_ref[...], w_ref[...]) * dinv


def _tmid_body(p0_ref, p1_ref, yp_ref, d0_ref, d1_ref, w_ref,
               rm_ref, rv_ref, ga_ref, be_ref, bc_ref, y_ref):
    dinv = _dinv_of(d0_ref[...], d1_ref[...])
    agg = (p0_ref[...] + p1_ref[...] + yp_ref[...]) * dinv + bc_ref[...]
    hb = (agg - rm_ref[...]) * lax.rsqrt(rv_ref[...] + 1e-5) * ga_ref[...] \
        + be_ref[...]
    h = jnp.maximum(hb, 0.0)
    y_ref[...] = _dot(h, w_ref[...]) * dinv


def _tfin_body(p0_ref, p1_ref, yp_ref, d0_ref, d1_ref, b_ref,
               rm_ref, rv_ref, ga_ref, be_ref, bc_ref,
               w1_ref, b1_ref, w2_ref, b2_ref, w3_ref, b3_ref,
               out_ref, pool_acc, cnt_acc):
    i = pl.program_id(0)
    nsteps = pl.num_programs(0)

    @pl.when(i == 0)
    def _init():
        pool_acc[...] = jnp.zeros_like(pool_acc)
        cnt_acc[...] = jnp.zeros_like(cnt_acc)

    dinv = _dinv_of(d0_ref[...], d1_ref[...])
    agg = (p0_ref[...] + p1_ref[...] + yp_ref[...]) * dinv + bc_ref[...]
    hb = (agg - rm_ref[...]) * lax.rsqrt(rv_ref[...] + 1e-5) * ga_ref[...] \
        + be_ref[...]
    h = jnp.maximum(hb, 0.0)

    gids = lax.broadcasted_iota(jnp.int32, (G, BM), 0)
    onehot = (b_ref[...] == gids).astype(jnp.float32)
    pool_acc[...] += _dot(onehot, h)
    cnt_acc[...] += jnp.broadcast_to(
        jnp.sum(onehot, axis=1, keepdims=True), (G, H))

    @pl.when(i == nsteps - 1)
    def _finish():
        pooled = pool_acc[...] / jnp.maximum(cnt_acc[...], 1.0)
        z = jnp.maximum(_dot(pooled, w1_ref[...]) + b1_ref[...], 0.0)
        z = jnp.maximum(_dot(z, w2_ref[...]) + b2_ref[...], 0.0)
        out_ref[...] = _dot(z, w3_ref[...]) + b3_ref[...]


def _row_spec(w):
    return pl.BlockSpec((BM, w), lambda i: (i, 0))


def _full_spec(r, c):
    return pl.BlockSpec((r, c), lambda i: (0, 0))


def kernel(x, edge_index, batch, Wc, bc, gamma, beta, rm, rv,
           Wm1, bm1, Wm2, bm2, Wm3, bm3):
    n, f = x.shape
    e = edge_index.shape[1]
    npad = _pad_up(n + 1, BM)
    ep = _pad_up(e, N_TILES * K * NB * 2)
    ch = ep // (N_TILES * K)
    ch_pair = ep // (N_SUB * K)
    ch0 = _pad_up(int(ch_pair * CH0_SHARE), 2 * NB)
    ch1 = ch_pair - ch0

    xp = jnp.zeros((npad, f), jnp.float32).at[:n].set(x)
    srcp = jnp.concatenate(
        [edge_index[0].astype(jnp.int32),
         jnp.zeros((ep - e,), jnp.int32)]).reshape(ep // K, K)
    dstp = jnp.concatenate(
        [edge_index[1].astype(jnp.int32),
         jnp.full((ep - e,), n, jnp.int32)]).reshape(ep // K, K)
    bpad = jnp.full((1, npad), G, jnp.int32).at[0, :n].set(
        batch.astype(jnp.int32))

    zeros_w = jnp.zeros((npad // N_SUB, H), jnp.float32)
    ones_w = jnp.ones((K, H), jnp.float32)

    w2p = jnp.zeros((H, H), jnp.float32).at[:, :Wm2.shape[1]].set(Wm2)
    b2p = jnp.zeros((1, H), jnp.float32).at[0, :Wm2.shape[1]].set(bm2)
    w3p = jnp.zeros((H, H), jnp.float32).at[:Wm3.shape[0], 0].set(Wm3[:, 0])
    b3p = jnp.zeros((1, H), jnp.float32).at[0, 0].set(bm3[0])
    w1p = Wm1
    b1p = bm1.reshape(1, H)

    params = [(rm[i].reshape(1, H), rv[i].reshape(1, H),
               gamma[i].reshape(1, H), beta[i].reshape(1, H),
               bc[i].reshape(1, H)) for i in range(3)]

    grid = (npad // BM,)

    degp = _deg_kernel(npad, ch)(dstp, ones_w, zeros_w)
    d0, d1 = degp[0], degp[1]

    y = pl.pallas_call(
        _t0_body,
        grid=grid,
        in_specs=[_row_spec(f), _full_spec(f, H), _row_spec(H), _row_spec(H)],
        out_specs=_row_spec(H),
        out_shape=jax.ShapeDtypeStruct((npad, H), jnp.float32),
    )(xp, Wc[0], d0, d1)

    for i in range(2):
        part = _scatter_kernel(npad, ch0, ch1)(y, srcp, dstp, zeros_w)
        rm_i, rv_i, ga_i, be_i, bc_i = params[i]
        y = pl.pallas_call(
            _tmid_body,
            grid=grid,
            in_specs=[_row_spec(H), _row_spec(H), _row_spec(H),
                      _row_spec(H), _row_spec(H), _full_spec(H, H),
                      _full_spec(1, H), _full_spec(1, H), _full_spec(1, H),
                      _full_spec(1, H), _full_spec(1, H)],
            out_specs=_row_spec(H),
            out_shape=jax.ShapeDtypeStruct((npad, H), jnp.float32),
        )(part[0], part[1], y, d0, d1, Wc[i + 1],
          rm_i, rv_i, ga_i, be_i, bc_i)

    part = _scatter_kernel(npad, ch0, ch1)(y, srcp, dstp, zeros_w)
    rm_i, rv_i, ga_i, be_i, bc_i = params[2]
    out = pl.pallas_call(
        _tfin_body,
        grid=grid,
        in_specs=[_row_spec(H), _row_spec(H), _row_spec(H),
                  _row_spec(H), _row_spec(H),
                  pl.BlockSpec((1, BM), lambda i: (0, i)),
                  _full_spec(1, H), _full_spec(1, H), _full_spec(1, H),
                  _full_spec(1, H), _full_spec(1, H),
                  _full_spec(H, H), _full_spec(1, H), _full_spec(H, H),
                  _full_spec(1, H), _full_spec(H, H), _full_spec(1, H)],
        out_specs=_full_spec(G, H),
        out_shape=jax.ShapeDtypeStruct((G, H), jnp.float32),
        scratch_shapes=[pltpu.VMEM((G, H), jnp.float32),
                        pltpu.VMEM((G, H), jnp.float32)],
    )(part[0], part[1], y, d0, d1, bpad,
      rm_i, rv_i, ga_i, be_i, bc_i,
      w1p, b1p, w2p, b2p, w3p, b3p)

    return out[:, 0]
